# baseline (speedup 1.0000x reference)
import math

import jax
import jax.numpy as jnp
from jax import lax
from jax.experimental import pallas as pl
from jax.experimental.pallas import tpu as pltpu
from jax.experimental.pallas import tpu_sc as plsc

NC = 2
NS = 16
CHUNK = 64
DCH = 128
EPS = 1e-5
F32 = jnp.float32


def _mesh():
    return plsc.VectorSubcoreMesh(
        core_axis_name="c", subcore_axis_name="s",
        num_cores=NC, num_subcores=NS)


def _make_deg(EP, NPAD):
    n_chunks = EP // (NC * NS * DCH)
    zrows = NPAD // NS

    W = 4

    def body(dst_hbm, p0_hbm, p1_hbm, hist, zbuf, ones, idst, sem):
        c = lax.axis_index("c")
        s = lax.axis_index("s")
        for i in range(zrows // 16):
            zbuf[pl.ds(i * 16, 16)] = jnp.zeros((16,), F32)
        for i in range(DCH // 16):
            ones[pl.ds(i * 16, 16)] = jnp.ones((16,), F32)
        w = c * NS + s
        pltpu.sync_copy(dst_hbm.at[pl.ds(w * n_chunks, n_chunks)], idst)
        pltpu.sync_copy(zbuf, hist.at[pl.ds(s * zrows, zrows)])
        plsc.subcore_barrier()

        def scat(j):
            pltpu.async_copy(ones, hist.at[idst.at[j]], sem, add=True)

        def drain(j):
            pltpu.make_async_copy(ones, hist.at[idst.at[j]], sem).wait()

        for j in range(W):
            scat(j)

        def step(j, carry):
            drain(j)
            scat(j + W)
            return carry

        lax.fori_loop(0, n_chunks - W, step, 0)
        for j in range(W):
            drain(n_chunks - W + j)
        plsc.subcore_barrier()

        @pl.when(c == 0)
        def _():
            pltpu.sync_copy(hist.at[pl.ds(s * zrows, zrows)],
                            p0_hbm.at[pl.ds(s * zrows, zrows)])

        @pl.when(c == 1)
        def _():
            pltpu.sync_copy(hist.at[pl.ds(s * zrows, zrows)],
                            p1_hbm.at[pl.ds(s * zrows, zrows)])

    return pl.kernel(
        body,
        out_type=(jax.ShapeDtypeStruct((NPAD,), F32),
                  jax.ShapeDtypeStruct((NPAD,), F32)),
        mesh=_mesh(),
        scratch_types=[
            pltpu.VMEM_SHARED((NPAD,), F32),
            pltpu.VMEM((zrows,), F32),
            pltpu.VMEM((DCH,), F32),
            pltpu.VMEM((n_chunks, DCH), jnp.int32),
            pltpu.SemaphoreType.DMA,
        ],
    )


def _edge_pipeline(y_hbm, isrc, idst, slab, bufs, n_chunks):
    def gather(j, b):
        pltpu.async_copy(y_hbm.at[isrc.at[pl.ds(j * CHUNK, CHUNK)]],
                         b[0], b[1])

    def wg(j, b):
        pltpu.make_async_copy(y_hbm.at[isrc.at[pl.ds(j * CHUNK, CHUNK)]],
                              b[0], b[1]).wait()

    def scatter(j, b):
        pltpu.async_copy(b[0], slab.at[idst.at[j]], b[2], add=True)

    def ws(j, b):
        pltpu.make_async_copy(b[0], slab.at[idst.at[j]], b[2]).wait()

    B0, B1, B2, B3 = bufs
    pairs = ((B0, B2), (B1, B3), (B2, B0), (B3, B1))

    gather(0, B0)
    gather(1, B1)
    wg(0, B0)
    scatter(0, B0)
    gather(2, B2)
    wg(1, B1)
    scatter(1, B1)
    gather(3, B3)

    def group4(c):
        for i, (bs, bg) in enumerate(pairs):
            wg(c + i + 2, bg)
            scatter(c + i + 2, bg)
            ws(c + i, bs)
            gather(c + i + 4, bs)

    G = n_chunks // 4 - 1

    def group8(k, carry):
        group4(8 * k)
        group4(8 * k + 4)
        return carry

    lax.fori_loop(0, G // 2, group8, 0)
    if G % 2 == 1:
        group4(4 * (G - 1))
    n = n_chunks
    ws(n - 4, B0)
    wg(n - 2, B2)
    scatter(n - 2, B2)
    ws(n - 3, B1)
    wg(n - 1, B3)
    scatter(n - 1, B3)
    ws(n - 2, B2)
    ws(n - 1, B3)


def _agg_scratch(D, NPAD, blk):
    return [
        pltpu.VMEM_SHARED((NPAD, D), F32),
        pltpu.VMEM((blk * CHUNK,), jnp.int32),
        pltpu.VMEM((blk // 2, DCH), jnp.int32),
        pltpu.VMEM((blk, CHUNK), jnp.int32),
        pltpu.VMEM((CHUNK, D), F32),
        pltpu.VMEM((CHUNK, D), F32),
        pltpu.VMEM((CHUNK, D), F32),
        pltpu.VMEM((CHUNK, D), F32),
        pltpu.SemaphoreType.DMA,
        pltpu.SemaphoreType.DMA,
        pltpu.SemaphoreType.DMA,
        pltpu.SemaphoreType.DMA,
        pltpu.SemaphoreType.DMA,
        pltpu.SemaphoreType.DMA,
        pltpu.SemaphoreType.DMA,
        pltpu.SemaphoreType.DMA,
    ]


def _blocked_edges(y_hbm, src_hbm, dst_hbm, slab, isrc, ista, idst, bufs,
                   row0, n_chunks, blk):
    def block(b, carry):
        off = pl.multiple_of(row0 + b * blk, 8)
        offd = pl.multiple_of((row0 // 2) + b * (blk // 2), 8)
        pltpu.sync_copy(src_hbm.at[pl.ds(off * CHUNK, blk * CHUNK)], isrc)
        pltpu.sync_copy(dst_hbm.at[pl.ds(offd, blk // 2)], ista)

        def repack(r, carry2):
            for k in range(DCH // 16):
                idst[2 * r + k // 4, pl.ds((k % 4) * 16, 16)] = (
                    ista[r, pl.ds(k * 16, 16)])
            return carry2

        lax.fori_loop(0, blk // 2, repack, 0)
        _edge_pipeline(y_hbm, isrc, idst, slab, bufs, blk)
        return carry

    lax.fori_loop(0, n_chunks // blk, block, 0)


def _make_agg(D, EP, NPAD, blk):
    n_chunks = EP // (NS * CHUNK)
    rpt = NPAD // NS

    def body(y0_hbm, y1_hbm, src_hbm, dst_hbm, a0_hbm, a1_hbm,
             slab, isrc, ista, idst, ra, rb, rc, rd,
             m0, m1, m2, m3, m4, m5, m6, m7):
        bufs = ((ra, m0, m1), (rb, m2, m3), (rc, m4, m5), (rd, m6, m7))
        c = lax.axis_index("c")
        s = lax.axis_index("s")
        r0 = s * rpt

        @pl.when(c == 0)
        def _():
            pltpu.sync_copy(y0_hbm.at[pl.ds(r0, rpt)],
                            slab.at[pl.ds(r0, rpt)])

        @pl.when(c == 1)
        def _():
            pltpu.sync_copy(y1_hbm.at[pl.ds(r0, rpt)],
                            slab.at[pl.ds(r0, rpt)])

        plsc.subcore_barrier()

        @pl.when(c == 0)
        def _():
            _blocked_edges(y0_hbm, src_hbm, dst_hbm, slab, isrc, ista,
                           idst, bufs, s * n_chunks, n_chunks, blk)

        @pl.when(c == 1)
        def _():
            _blocked_edges(y1_hbm, src_hbm, dst_hbm, slab, isrc, ista,
                           idst, bufs, s * n_chunks, n_chunks, blk)

        plsc.subcore_barrier()

        @pl.when(c == 0)
        def _():
            pltpu.sync_copy(slab.at[pl.ds(r0, rpt)],
                            a0_hbm.at[pl.ds(r0, rpt)])

        @pl.when(c == 1)
        def _():
            pltpu.sync_copy(slab.at[pl.ds(r0, rpt)],
                            a1_hbm.at[pl.ds(r0, rpt)])

    return pl.kernel(
        body,
        out_type=(jax.ShapeDtypeStruct((NPAD, D), F32),
                  jax.ShapeDtypeStruct((NPAD, D), F32)),
        mesh=_mesh(),
        scratch_types=_agg_scratch(D, NPAD, blk),
    )


def _make_agg2(D, EP, NPAD, blk):
    n_chunks = EP // (NC * NS * CHUNK)
    rpt = NPAD // NS

    def body(y_hbm, src_hbm, dst_hbm, s0_hbm, s1_hbm,
             slab, isrc, ista, idst, ra, rb, rc, rd,
             m0, m1, m2, m3, m4, m5, m6, m7):
        bufs = ((ra, m0, m1), (rb, m2, m3), (rc, m4, m5), (rd, m6, m7))
        c = lax.axis_index("c")
        s = lax.axis_index("s")
        r0 = s * rpt
        w = c * NS + s

        pltpu.sync_copy(y_hbm.at[pl.ds(r0, rpt)], slab.at[pl.ds(r0, rpt)])
        plsc.subcore_barrier()

        _blocked_edges(y_hbm, src_hbm, dst_hbm, slab, isrc, ista,
                       idst, bufs, w * n_chunks, n_chunks, blk)
        plsc.subcore_barrier()

        @pl.when(c == 0)
        def _():
            pltpu.sync_copy(slab.at[pl.ds(r0, rpt)],
                            s0_hbm.at[pl.ds(r0, rpt)])

        @pl.when(c == 1)
        def _():
            pltpu.sync_copy(slab.at[pl.ds(r0, rpt)],
                            s1_hbm.at[pl.ds(r0, rpt)])

    return pl.kernel(
        body,
        out_type=(jax.ShapeDtypeStruct((NPAD, D), F32),
                  jax.ShapeDtypeStruct((NPAD, D), F32)),
        mesh=_mesh(),
        scratch_types=_agg_scratch(D, NPAD, blk),
    )


def _tc_prep(N, DIN, DH, NPAD):
    def body(x_ref, w_ref, p0_ref, p1_ref, dis_ref, y0_ref, y1_ref):
        deg = p0_ref[0:N, :] + p1_ref[0:N, :] + 1.0
        dis = lax.rsqrt(deg)
        dis_ref[...] = dis
        z = jnp.dot(x_ref[...], w_ref[...], preferred_element_type=F32)
        z = z * dis
        y0_ref[0:N, :] = z[:, :DH // 2]
        y1_ref[0:N, :] = z[:, DH // 2:]
        y0_ref[N:NPAD, :] = jnp.zeros((NPAD - N, DH // 2), F32)
        y1_ref[N:NPAD, :] = jnp.zeros((NPAD - N, DH // 2), F32)

    return pl.pallas_call(
        body,
        out_shape=(jax.ShapeDtypeStruct((N, 1), F32),
                   jax.ShapeDtypeStruct((NPAD, DH // 2), F32),
                   jax.ShapeDtypeStruct((NPAD, DH // 2), F32)),
    )


def _tc_mid(N, DH, DOUT, NPAD):
    def body(a0_ref, a1_ref, dis_ref, b_ref, g_ref, be_ref, w_ref,
             o0_ref):
        dis = dis_ref[...]
        t = jnp.concatenate([a0_ref[0:N, :], a1_ref[0:N, :]], axis=1)
        t = t * dis + b_ref[...]
        mean = jnp.mean(t, axis=0, keepdims=True)
        var = jnp.mean((t - mean) ** 2, axis=0, keepdims=True)
        h = (t - mean) * lax.rsqrt(var + EPS) * g_ref[...] + be_ref[...]
        h = jnp.maximum(h, 0.0)
        z = jnp.dot(h, w_ref[...], preferred_element_type=F32)
        z = z * dis
        o0_ref[0:N, :] = z
        o0_ref[N:NPAD, :] = jnp.zeros((NPAD - N, DOUT), F32)

    return pl.pallas_call(
        body,
        out_shape=jax.ShapeDtypeStruct((NPAD, DOUT), F32),
    )


def _tc_final(N, DOUT):
    def body(s0_ref, s1_ref, y_ref, dis_ref, b_ref, g_ref, be_ref, o_ref):
        t = s0_ref[0:N, :] + s1_ref[0:N, :] - y_ref[0:N, :]
        t = t * dis_ref[...] + b_ref[...]
        mean = jnp.mean(t, axis=0, keepdims=True)
        var = jnp.mean((t - mean) ** 2, axis=0, keepdims=True)
        o_ref[...] = (t - mean) * lax.rsqrt(var + EPS) * g_ref[...] + be_ref[...]

    return pl.pallas_call(
        body,
        out_shape=jax.ShapeDtypeStruct((N, DOUT), F32),
    )


def kernel(x, edge_index, W1, b1, g1, be1, W2, b2, g2, be2):
    N, DIN = x.shape
    DH = W1.shape[1]
    DOUT = W2.shape[1]
    E = edge_index.shape[1]

    blk1, blk2 = 64, 32
    ealign = math.lcm(NS * CHUNK * blk1, NC * NS * CHUNK * blk2,
                      2 * NC * NS * DCH)
    EP = -(-E // ealign) * ealign
    NPAD = (N // 2048 + 1) * 2048

    src = edge_index[0]
    dst = edge_index[1]
    pad_i = jnp.arange(EP - E, dtype=jnp.int32)
    srcp = jnp.concatenate([src, (pad_i * 97) % N])
    dstp = jnp.concatenate([dst, N + pad_i % (NPAD - N)]).reshape(EP // DCH, DCH)

    p0, p1 = _make_deg(EP, NPAD)(dstp)
    p0 = p0.reshape(NPAD, 1)
    p1 = p1.reshape(NPAD, 1)
    dis, y10, y11 = _tc_prep(N, DIN, DH, NPAD)(x, W1, p0, p1)
    a10, a11 = _make_agg(DH // 2, EP, NPAD, blk1)(y10, y11, srcp, dstp)
    y2 = _tc_mid(N, DH, DOUT, NPAD)(a10, a11, dis, b1, g1, be1, W2)
    s20, s21 = _make_agg2(DOUT, EP, NPAD, blk2)(y2, srcp, dstp)
    return _tc_final(N, DOUT)(s20, s21, y2, dis, b2, g2, be2)


# speedup vs baseline: 1.0024x; 1.0024x over previous
import math

import jax
import jax.numpy as jnp
from jax import lax
from jax.experimental import pallas as pl
from jax.experimental.pallas import tpu as pltpu
from jax.experimental.pallas import tpu_sc as plsc

NC = 2
NS = 16
CHUNK = 64
DCH = 128
EPS = 1e-5
F32 = jnp.float32


def _mesh():
    return plsc.VectorSubcoreMesh(
        core_axis_name="c", subcore_axis_name="s",
        num_cores=NC, num_subcores=NS)


def _make_deg(EP, NPAD):
    n_chunks = EP // (NC * NS * DCH)
    zrows = NPAD // NS

    W = 4

    def body(dst_hbm, p0_hbm, p1_hbm, hist, zbuf, ones, idst, sem):
        c = lax.axis_index("c")
        s = lax.axis_index("s")
        for i in range(zrows // 16):
            zbuf[pl.ds(i * 16, 16)] = jnp.zeros((16,), F32)
        for i in range(DCH // 16):
            ones[pl.ds(i * 16, 16)] = jnp.ones((16,), F32)
        w = c * NS + s
        pltpu.sync_copy(dst_hbm.at[pl.ds(w * n_chunks, n_chunks)], idst)
        pltpu.sync_copy(zbuf, hist.at[pl.ds(s * zrows, zrows)])
        plsc.subcore_barrier()

        def scat(j):
            pltpu.async_copy(ones, hist.at[idst.at[j]], sem, add=True)

        def drain(j):
            pltpu.make_async_copy(ones, hist.at[idst.at[j]], sem).wait()

        for j in range(W):
            scat(j)

        def step(j, carry):
            drain(j)
            scat(j + W)
            return carry

        lax.fori_loop(0, n_chunks - W, step, 0)
        for j in range(W):
            drain(n_chunks - W + j)
        plsc.subcore_barrier()

        @pl.when(c == 0)
        def _():
            pltpu.sync_copy(hist.at[pl.ds(s * zrows, zrows)],
                            p0_hbm.at[pl.ds(s * zrows, zrows)])

        @pl.when(c == 1)
        def _():
            pltpu.sync_copy(hist.at[pl.ds(s * zrows, zrows)],
                            p1_hbm.at[pl.ds(s * zrows, zrows)])

    return pl.kernel(
        body,
        out_type=(jax.ShapeDtypeStruct((NPAD,), F32),
                  jax.ShapeDtypeStruct((NPAD,), F32)),
        mesh=_mesh(),
        scratch_types=[
            pltpu.VMEM_SHARED((NPAD,), F32),
            pltpu.VMEM((zrows,), F32),
            pltpu.VMEM((DCH,), F32),
            pltpu.VMEM((n_chunks, DCH), jnp.int32),
            pltpu.SemaphoreType.DMA,
        ],
    )


def _edge_pipeline(y_hbm, isrc, idst, slab, bufs, n_chunks):
    def gather(j, b):
        pltpu.async_copy(y_hbm.at[isrc.at[pl.ds(j * CHUNK, CHUNK)]],
                         b[0], b[1])

    def wg(j, b):
        pltpu.make_async_copy(y_hbm.at[isrc.at[pl.ds(j * CHUNK, CHUNK)]],
                              b[0], b[1]).wait()

    def scatter(j, b):
        pltpu.async_copy(b[0], slab.at[idst.at[j]], b[2], add=True)

    def ws(j, b):
        pltpu.make_async_copy(b[0], slab.at[idst.at[j]], b[2]).wait()

    B0, B1, B2, B3 = bufs
    pairs = ((B0, B2), (B1, B3), (B2, B0), (B3, B1))

    gather(0, B0)
    gather(1, B1)
    wg(0, B0)
    scatter(0, B0)
    gather(2, B2)
    wg(1, B1)
    scatter(1, B1)
    gather(3, B3)

    def group(k, carry):
        c = 4 * k
        for i, (bs, bg) in enumerate(pairs):
            wg(c + i + 2, bg)
            scatter(c + i + 2, bg)
            ws(c + i, bs)
            gather(c + i + 4, bs)
        return carry

    lax.fori_loop(0, n_chunks // 4 - 1, group, 0)
    n = n_chunks
    ws(n - 4, B0)
    wg(n - 2, B2)
    scatter(n - 2, B2)
    ws(n - 3, B1)
    wg(n - 1, B3)
    scatter(n - 1, B3)
    ws(n - 2, B2)
    ws(n - 1, B3)


def _agg_scratch(D, NPAD, blk):
    return [
        pltpu.VMEM_SHARED((NPAD, D), F32),
        pltpu.VMEM((blk * CHUNK,), jnp.int32),
        pltpu.VMEM((blk // 2, DCH), jnp.int32),
        pltpu.VMEM((blk, CHUNK), jnp.int32),
        pltpu.VMEM((CHUNK, D), F32),
        pltpu.VMEM((CHUNK, D), F32),
        pltpu.VMEM((CHUNK, D), F32),
        pltpu.VMEM((CHUNK, D), F32),
        pltpu.SemaphoreType.DMA,
        pltpu.SemaphoreType.DMA,
        pltpu.SemaphoreType.DMA,
        pltpu.SemaphoreType.DMA,
        pltpu.SemaphoreType.DMA,
        pltpu.SemaphoreType.DMA,
        pltpu.SemaphoreType.DMA,
        pltpu.SemaphoreType.DMA,
    ]


def _blocked_edges(y_hbm, src_hbm, dst_hbm, slab, isrc, ista, idst, bufs,
                   row0, n_chunks, blk):
    def block(b, carry):
        off = pl.multiple_of(row0 + b * blk, 8)
        offd = pl.multiple_of((row0 // 2) + b * (blk // 2), 8)
        pltpu.sync_copy(src_hbm.at[pl.ds(off * CHUNK, blk * CHUNK)], isrc)
        pltpu.sync_copy(dst_hbm.at[pl.ds(offd, blk // 2)], ista)

        def repack(r, carry2):
            for k in range(DCH // 16):
                idst[2 * r + k // 4, pl.ds((k % 4) * 16, 16)] = (
                    ista[r, pl.ds(k * 16, 16)])
            return carry2

        lax.fori_loop(0, blk // 2, repack, 0)
        _edge_pipeline(y_hbm, isrc, idst, slab, bufs, blk)
        return carry

    lax.fori_loop(0, n_chunks // blk, block, 0)


def _make_agg(D, EP, NPAD, blk):
    n_chunks = EP // (NS * CHUNK)
    rpt = NPAD // NS

    def body(y0_hbm, y1_hbm, src_hbm, dst_hbm, a0_hbm, a1_hbm,
             slab, isrc, ista, idst, ra, rb, rc, rd,
             m0, m1, m2, m3, m4, m5, m6, m7):
        bufs = ((ra, m0, m1), (rb, m2, m3), (rc, m4, m5), (rd, m6, m7))
        c = lax.axis_index("c")
        s = lax.axis_index("s")
        r0 = s * rpt

        @pl.when(c == 0)
        def _():
            pltpu.sync_copy(y0_hbm.at[pl.ds(r0, rpt)],
                            slab.at[pl.ds(r0, rpt)])

        @pl.when(c == 1)
        def _():
            pltpu.sync_copy(y1_hbm.at[pl.ds(r0, rpt)],
                            slab.at[pl.ds(r0, rpt)])

        plsc.subcore_barrier()

        @pl.when(c == 0)
        def _():
            _blocked_edges(y0_hbm, src_hbm, dst_hbm, slab, isrc, ista,
                           idst, bufs, s * n_chunks, n_chunks, blk)

        @pl.when(c == 1)
        def _():
            _blocked_edges(y1_hbm, src_hbm, dst_hbm, slab, isrc, ista,
                           idst, bufs, s * n_chunks, n_chunks, blk)

        plsc.subcore_barrier()

        @pl.when(c == 0)
        def _():
            pltpu.sync_copy(slab.at[pl.ds(r0, rpt)],
                            a0_hbm.at[pl.ds(r0, rpt)])

        @pl.when(c == 1)
        def _():
            pltpu.sync_copy(slab.at[pl.ds(r0, rpt)],
                            a1_hbm.at[pl.ds(r0, rpt)])

    return pl.kernel(
        body,
        out_type=(jax.ShapeDtypeStruct((NPAD, D), F32),
                  jax.ShapeDtypeStruct((NPAD, D), F32)),
        mesh=_mesh(),
        scratch_types=_agg_scratch(D, NPAD, blk),
    )


def _make_agg2(D, EP, NPAD, blk):
    n_chunks = EP // (NC * NS * CHUNK)
    rpt = NPAD // NS

    def body(y_hbm, src_hbm, dst_hbm, s0_hbm, s1_hbm,
             slab, isrc, ista, idst, ra, rb, rc, rd,
             m0, m1, m2, m3, m4, m5, m6, m7):
        bufs = ((ra, m0, m1), (rb, m2, m3), (rc, m4, m5), (rd, m6, m7))
        c = lax.axis_index("c")
        s = lax.axis_index("s")
        r0 = s * rpt
        w = c * NS + s

        pltpu.sync_copy(y_hbm.at[pl.ds(r0, rpt)], slab.at[pl.ds(r0, rpt)])
        plsc.subcore_barrier()

        _blocked_edges(y_hbm, src_hbm, dst_hbm, slab, isrc, ista,
                       idst, bufs, w * n_chunks, n_chunks, blk)
        plsc.subcore_barrier()

        @pl.when(c == 0)
        def _():
            pltpu.sync_copy(slab.at[pl.ds(r0, rpt)],
                            s0_hbm.at[pl.ds(r0, rpt)])

        @pl.when(c == 1)
        def _():
            pltpu.sync_copy(slab.at[pl.ds(r0, rpt)],
                            s1_hbm.at[pl.ds(r0, rpt)])

    return pl.kernel(
        body,
        out_type=(jax.ShapeDtypeStruct((NPAD, D), F32),
                  jax.ShapeDtypeStruct((NPAD, D), F32)),
        mesh=_mesh(),
        scratch_types=_agg_scratch(D, NPAD, blk),
    )


def _tc_prep(N, DIN, DH, NPAD):
    def body(x_ref, w_ref, p0_ref, p1_ref, dis_ref, y0_ref, y1_ref):
        deg = p0_ref[0:N, :] + p1_ref[0:N, :] + 1.0
        dis = lax.rsqrt(deg)
        dis_ref[...] = dis
        z = jnp.dot(x_ref[...], w_ref[...], preferred_element_type=F32)
        z = z * dis
        y0_ref[0:N, :] = z[:, :DH // 2]
        y1_ref[0:N, :] = z[:, DH // 2:]
        y0_ref[N:NPAD, :] = jnp.zeros((NPAD - N, DH // 2), F32)
        y1_ref[N:NPAD, :] = jnp.zeros((NPAD - N, DH // 2), F32)

    return pl.pallas_call(
        body,
        out_shape=(jax.ShapeDtypeStruct((N, 1), F32),
                   jax.ShapeDtypeStruct((NPAD, DH // 2), F32),
                   jax.ShapeDtypeStruct((NPAD, DH // 2), F32)),
    )


def _tc_mid(N, DH, DOUT, NPAD):
    def body(a0_ref, a1_ref, dis_ref, b_ref, g_ref, be_ref, w_ref,
             o0_ref):
        dis = dis_ref[...]
        t = jnp.concatenate([a0_ref[0:N, :], a1_ref[0:N, :]], axis=1)
        t = t * dis + b_ref[...]
        mean = jnp.mean(t, axis=0, keepdims=True)
        var = jnp.mean((t - mean) ** 2, axis=0, keepdims=True)
        h = (t - mean) * lax.rsqrt(var + EPS) * g_ref[...] + be_ref[...]
        h = jnp.maximum(h, 0.0)
        z = jnp.dot(h, w_ref[...], preferred_element_type=F32)
        z = z * dis
        o0_ref[0:N, :] = z
        o0_ref[N:NPAD, :] = jnp.zeros((NPAD - N, DOUT), F32)

    return pl.pallas_call(
        body,
        out_shape=jax.ShapeDtypeStruct((NPAD, DOUT), F32),
    )


def _tc_final(N, DOUT):
    def body(s0_ref, s1_ref, y_ref, dis_ref, b_ref, g_ref, be_ref, o_ref):
        t = s0_ref[0:N, :] + s1_ref[0:N, :] - y_ref[0:N, :]
        t = t * dis_ref[...] + b_ref[...]
        mean = jnp.mean(t, axis=0, keepdims=True)
        var = jnp.mean((t - mean) ** 2, axis=0, keepdims=True)
        o_ref[...] = (t - mean) * lax.rsqrt(var + EPS) * g_ref[...] + be_ref[...]

    return pl.pallas_call(
        body,
        out_shape=jax.ShapeDtypeStruct((N, DOUT), F32),
    )


def kernel(x, edge_index, W1, b1, g1, be1, W2, b2, g2, be2):
    N, DIN = x.shape
    DH = W1.shape[1]
    DOUT = W2.shape[1]
    E = edge_index.shape[1]

    blk1, blk2 = 64, 32
    ealign = math.lcm(NS * CHUNK * blk1, NC * NS * CHUNK * blk2,
                      2 * NC * NS * DCH)
    EP = -(-E // ealign) * ealign
    NPAD = (N // 2048 + 1) * 2048

    src = edge_index[0]
    dst = edge_index[1]
    pad_i = jnp.arange(EP - E, dtype=jnp.int32)
    srcp = jnp.concatenate([src, (pad_i * 97) % N])
    dstp = jnp.concatenate([dst, N + pad_i % (NPAD - N)]).reshape(EP // DCH, DCH)

    p0, p1 = _make_deg(EP, NPAD)(dstp)
    p0 = p0.reshape(NPAD, 1)
    p1 = p1.reshape(NPAD, 1)
    dis, y10, y11 = _tc_prep(N, DIN, DH, NPAD)(x, W1, p0, p1)
    a10, a11 = _make_agg(DH // 2, EP, NPAD, blk1)(y10, y11, srcp, dstp)
    y2 = _tc_mid(N, DH, DOUT, NPAD)(a10, a11, dis, b1, g1, be1, W2)
    s20, s21 = _make_agg2(DOUT, EP, NPAD, blk2)(y2, srcp, dstp)
    return _tc_final(N, DOUT)(s20, s21, y2, dis, b2, g2, be2)
